# baseline (speedup 1.0000x reference)
import functools

import jax
import jax.numpy as jnp
import numpy as np
from jax import lax
from jax.experimental import pallas as pl
from jax.experimental.pallas import tpu as pltpu, tpu_sc as plsc

_N = 10000
_D = 128
_G = 128
_OUT = 10

_CH = 128
_NCHUNK = 80
_EPT = _CH * _NCHUNK
_NW = 32
_EPAD = _EPT * _NW
_NACC = 10240
_ZROWS = _NACC // 16
_HC = _NCHUNK // 2

def _agg_body(table_hbm, src_hbm, dst_hbm, zeros_hbm, out_hbm,
              src_v, dst_v, rows0_v, rows1_v, acc_sh, sem0, sem1, semz):
    c = lax.axis_index("c")
    s = lax.axis_index("s")
    wid = s * 2 + c
    pltpu.async_copy(zeros_hbm.at[pl.ds(s * _ZROWS, _ZROWS)],
                     acc_sh.at[pl.ds(s * _ZROWS, _ZROWS)], semz)

    base = wid * _EPT
    for half in range(2):
        hoff = base + half * (_HC * _CH)
        pltpu.sync_copy(src_hbm.at[pl.ds(hoff, _HC * _CH)], src_v)
        pltpu.sync_copy(dst_hbm.at[pl.ds(hoff, _HC * _CH)], dst_v)
        pltpu.async_copy(table_hbm.at[src_v.at[pl.ds(0, _CH)]], rows0_v, sem0)
        pltpu.async_copy(table_hbm.at[src_v.at[pl.ds(_CH, _CH)]], rows1_v, sem1)
        if half == 0:
            pltpu.make_async_copy(
                zeros_hbm.at[pl.ds(s * _ZROWS, _ZROWS)],
                acc_sh.at[pl.ds(s * _ZROWS, _ZROWS)], semz).wait()
            plsc.subcore_barrier()

        def body(i, carry):
            o0 = i * 2 * _CH
            pltpu.make_async_copy(table_hbm.at[src_v.at[pl.ds(o0, _CH)]], rows0_v, sem0).wait()
            pltpu.sync_copy(rows0_v, acc_sh.at[dst_v.at[pl.ds(o0, _CH)]], add=True)
            pltpu.async_copy(table_hbm.at[src_v.at[pl.ds(o0 + 2 * _CH, _CH)]], rows0_v, sem0)
            pltpu.make_async_copy(table_hbm.at[src_v.at[pl.ds(o0 + _CH, _CH)]], rows1_v, sem1).wait()
            pltpu.sync_copy(rows1_v, acc_sh.at[dst_v.at[pl.ds(o0 + _CH, _CH)]], add=True)
            pltpu.async_copy(table_hbm.at[src_v.at[pl.ds(o0 + 3 * _CH, _CH)]], rows1_v, sem1)
            return carry

        lax.fori_loop(0, _HC // 2 - 1, body, 0)
        ot = (_HC - 2) * _CH
        pltpu.make_async_copy(table_hbm.at[src_v.at[pl.ds(ot, _CH)]], rows0_v, sem0).wait()
        pltpu.sync_copy(rows0_v, acc_sh.at[dst_v.at[pl.ds(ot, _CH)]], add=True)
        pltpu.make_async_copy(table_hbm.at[src_v.at[pl.ds(ot + _CH, _CH)]], rows1_v, sem1).wait()
        pltpu.sync_copy(rows1_v, acc_sh.at[dst_v.at[pl.ds(ot + _CH, _CH)]], add=True)
    plsc.subcore_barrier()
    pltpu.sync_copy(acc_sh.at[pl.ds(s * _ZROWS, _ZROWS)],
                    out_hbm.at[c, pl.ds(s * _ZROWS, _ZROWS)])


@functools.cache
def _get_agg():
    mesh = plsc.VectorSubcoreMesh(core_axis_name="c", subcore_axis_name="s")
    return pl.kernel(
        _agg_body,
        out_type=jax.ShapeDtypeStruct((2, _NACC, _D), jnp.float32),
        mesh=mesh,
        scratch_types=[
            pltpu.VMEM((_HC * _CH,), jnp.int32),
            pltpu.VMEM((_HC * _CH,), jnp.int32),
            pltpu.VMEM((_CH, _D), jnp.float32),
            pltpu.VMEM((_CH, _D), jnp.float32),
            pltpu.VMEM_SHARED((_NACC, _D), jnp.float32),
            pltpu.SemaphoreType.DMA,
            pltpu.SemaphoreType.DMA,
            pltpu.SemaphoreType.DMA,
        ],
    )


def _agg_kernel(table, src, dst, zeros):
    return _get_agg()(table, src, dst, zeros)


def _mm(a, w):
    return jnp.dot(a, w, preferred_element_type=jnp.float32)


def _dense1_body(x_ref, p_ref, w1, b1, w2, b2, w3, b3, o_ref):
    h = x_ref[...] + p_ref[0] + p_ref[1]
    h = jnp.maximum(_mm(h, w1[...]) + b1[...], 0.0)
    h = jnp.maximum(_mm(h, w2[...]) + b2[...], 0.0)
    h = _mm(h, w3[...]) + b3[...]
    o_ref[...] = jnp.maximum(h, 0.0)


def _dense2_body(x_ref, p_ref, w1, b1, w2, b2, w3, b3, w4, b4, fw, fb, o_ref):
    h = x_ref[...] + p_ref[0] + p_ref[1]
    h = jnp.maximum(_mm(h, w1[...]) + b1[...], 0.0)
    h = jnp.maximum(_mm(h, w2[...]) + b2[...], 0.0)
    h = jnp.maximum(_mm(h, w3[...]) + b3[...], 0.0)
    h = _mm(h, w4[...]) + b4[...]
    o_ref[...] = jnp.maximum(_mm(h, fw[...]) + fb[...], 0.0)


def _dense3_body(nblk, x_ref, p_ref, batch_ref,
                 w1, b1, w2, b2, w3, b3, fw2, fb2, fw3, fb3,
                 o_ref, acc_ref):
    k = pl.program_id(0)

    @pl.when(k == 0)
    def _():
        acc_ref[...] = jnp.zeros_like(acc_ref)

    h = x_ref[...] + p_ref[0] + p_ref[1]
    h = jnp.maximum(_mm(h, w1[...]) + b1[...], 0.0)
    h = jnp.maximum(_mm(h, w2[...]) + b2[...], 0.0)
    h = _mm(h, w3[...]) + b3[...]
    r = h.shape[0]
    gi = lax.broadcasted_iota(jnp.int32, (_G, r), 0)
    m = (gi == batch_ref[0]).astype(jnp.float32)
    acc_ref[...] += _mm(m, h)

    @pl.when(k == nblk - 1)
    def _():
        z = jnp.maximum(_mm(acc_ref[...], fw2[...]) + fb2[...], 0.0)
        logits = _mm(z, fw3[...]) + fb3[...]
        mx = jnp.max(logits, axis=1, keepdims=True)
        lse = jnp.log(jnp.sum(jnp.exp(logits - mx), axis=1, keepdims=True)) + mx
        o_ref[...] = logits - lse


_R = 2000
_KB = _N // _R

_wspec = pl.BlockSpec((_D, _D), lambda i: (0, 0))
_bspec = pl.BlockSpec((1, _D), lambda i: (0, 0))
_rspec = pl.BlockSpec((_R, _D), lambda i: (i, 0))
_pspec = pl.BlockSpec((2, _R, _D), lambda i: (0, i, 0))


def _dense1(x, parts, w1, b1, w2, b2, w3, b3):
    return pl.pallas_call(
        _dense1_body,
        grid=(_KB,),
        in_specs=[_rspec, _pspec, _wspec, _bspec, _wspec, _bspec, _wspec, _bspec],
        out_specs=_rspec,
        out_shape=jax.ShapeDtypeStruct((_N, _D), jnp.float32),
    )(x, parts, w1, b1, w2, b2, w3, b3)


def _dense2(x, parts, w1, b1, w2, b2, w3, b3, w4, b4, fw, fb):
    return pl.pallas_call(
        _dense2_body,
        grid=(_KB,),
        in_specs=[_rspec, _pspec] + [_wspec, _bspec] * 5,
        out_specs=_rspec,
        out_shape=jax.ShapeDtypeStruct((_N, _D), jnp.float32),
    )(x, parts, w1, b1, w2, b2, w3, b3, w4, b4, fw, fb)


def _dense3(x, parts, batch3, w1, b1, w2, b2, w3, b3, fw2, fb2, fw3p, fb3p):
    return pl.pallas_call(
        functools.partial(_dense3_body, _KB),
        grid=(_KB,),
        in_specs=[_rspec, _pspec, pl.BlockSpec((1, 1, _R), lambda i: (i, 0, 0))]
                 + [_wspec, _bspec] * 5,
        out_specs=pl.BlockSpec((_G, _D), lambda i: (0, 0)),
        out_shape=jax.ShapeDtypeStruct((_G, _D), jnp.float32),
        scratch_shapes=[pltpu.VMEM((_G, _D), jnp.float32)],
    )(x, parts, batch3, w1, b1, w2, b2, w3, b3, fw2, fb2, fw3p, fb3p)


def kernel(x, edge_index, batch,
           c1w1, c1b1, c1w2, c1b2, c1w3, c1b3,
           c2w1, c2b1, c2w2, c2b2, c2w3, c2b3, c2w4, c2b4,
           c3w1, c3b1, c3w2, c3b2, c3w3, c3b3,
           fc1w, fc1b, fc2w, fc2b, fc3w, fc3b):
    e = edge_index.shape[1]
    pad = _EPAD - e
    pad_i = np.arange(pad, dtype=np.int32)
    src = jnp.concatenate([edge_index[0], pad_i % _N])
    dst = jnp.concatenate([edge_index[1], _N + pad_i % (_NACC - _N)])
    zeros = jnp.asarray(np.zeros((_NACC, _D), np.float32))
    batch3 = batch.reshape(_KB, 1, _R)

    def r2(b):
        return b.reshape(1, _D)

    fw3p = jnp.zeros((_D, _D), jnp.float32).at[:, :_OUT].set(fc3w)
    fb3p = jnp.full((1, _D), -1e9, jnp.float32).at[0, :_OUT].set(fc3b)

    p1 = _agg_kernel(x, src, dst, zeros)
    h1 = _dense1(x, p1, c1w1, r2(c1b1), c1w2, r2(c1b2), c1w3, r2(c1b3))
    p2 = _agg_kernel(h1, src, dst, zeros)
    h2 = _dense2(h1, p2, c2w1, r2(c2b1), c2w2, r2(c2b2), c2w3, r2(c2b3),
                 c2w4, r2(c2b4), fc1w, r2(fc1b))
    p3 = _agg_kernel(h2, src, dst, zeros)
    res = _dense3(h2, p3, batch3, c3w1, r2(c3b1), c3w2, r2(c3b2),
                  c3w3, r2(c3b3), fc2w, r2(fc2b), fw3p, fb3p)
    return res[:, :_OUT]


# speedup vs baseline: 1.0254x; 1.0254x over previous
import functools

import jax
import jax.numpy as jnp
import numpy as np
from jax import lax
from jax.experimental import pallas as pl
from jax.experimental.pallas import tpu as pltpu, tpu_sc as plsc

_N = 10000
_D = 128
_G = 128
_OUT = 10

_CH = 128
_NCHUNK = 80
_EPT = _CH * _NCHUNK
_NW = 32
_EPAD = _EPT * _NW
_NACC = 10240
_ZROWS = _NACC // 16
_HC = _NCHUNK // 2
_NROW = _EPAD // _CH
_E = 320000
_DK = 5
_DR = _NROW // _DK
_DE = _DR * _CH

def _agg_body(table_hbm, src_hbm, dst_hbm, zeros_hbm, out_hbm,
              src_v, dst_v, rows0_v, rows1_v, acc_sh, sem0, sem1, semz):
    c = lax.axis_index("c")
    s = lax.axis_index("s")
    wid = s * 2 + c
    pltpu.async_copy(zeros_hbm.at[pl.ds(s * _ZROWS, _ZROWS)],
                     acc_sh.at[pl.ds(s * _ZROWS, _ZROWS)], semz)

    for half in range(2):
        roff = wid * _NCHUNK + half * _HC
        pltpu.sync_copy(src_hbm.at[pl.ds(roff, _HC)], src_v)
        pltpu.sync_copy(dst_hbm.at[pl.ds(roff, _HC)], dst_v)
        pltpu.async_copy(table_hbm.at[src_v.at[0]], rows0_v, sem0)
        pltpu.async_copy(table_hbm.at[src_v.at[1]], rows1_v, sem1)
        if half == 0:
            pltpu.make_async_copy(
                zeros_hbm.at[pl.ds(s * _ZROWS, _ZROWS)],
                acc_sh.at[pl.ds(s * _ZROWS, _ZROWS)], semz).wait()
            plsc.subcore_barrier()

        def body(i, carry):
            j0 = i * 2
            pltpu.make_async_copy(table_hbm.at[src_v.at[j0]], rows0_v, sem0).wait()
            pltpu.sync_copy(rows0_v, acc_sh.at[dst_v.at[j0]], add=True)
            pltpu.async_copy(table_hbm.at[src_v.at[j0 + 2]], rows0_v, sem0)
            pltpu.make_async_copy(table_hbm.at[src_v.at[j0 + 1]], rows1_v, sem1).wait()
            pltpu.sync_copy(rows1_v, acc_sh.at[dst_v.at[j0 + 1]], add=True)
            pltpu.async_copy(table_hbm.at[src_v.at[j0 + 3]], rows1_v, sem1)
            return carry

        lax.fori_loop(0, _HC // 2 - 1, body, 0)
        jt = _HC - 2
        pltpu.make_async_copy(table_hbm.at[src_v.at[jt]], rows0_v, sem0).wait()
        pltpu.sync_copy(rows0_v, acc_sh.at[dst_v.at[jt]], add=True)
        pltpu.make_async_copy(table_hbm.at[src_v.at[jt + 1]], rows1_v, sem1).wait()
        pltpu.sync_copy(rows1_v, acc_sh.at[dst_v.at[jt + 1]], add=True)
    plsc.subcore_barrier()
    pltpu.sync_copy(acc_sh.at[pl.ds(s * _ZROWS, _ZROWS)],
                    out_hbm.at[c, pl.ds(s * _ZROWS, _ZROWS)])


@functools.cache
def _get_agg():
    mesh = plsc.VectorSubcoreMesh(core_axis_name="c", subcore_axis_name="s")
    return pl.kernel(
        _agg_body,
        out_type=jax.ShapeDtypeStruct((2, _NACC, _D), jnp.float32),
        mesh=mesh,
        scratch_types=[
            pltpu.VMEM((_HC, _CH), jnp.int32),
            pltpu.VMEM((_HC, _CH), jnp.int32),
            pltpu.VMEM((_CH, _D), jnp.float32),
            pltpu.VMEM((_CH, _D), jnp.float32),
            pltpu.VMEM_SHARED((_NACC, _D), jnp.float32),
            pltpu.SemaphoreType.DMA,
            pltpu.SemaphoreType.DMA,
            pltpu.SemaphoreType.DMA,
        ],
    )


def _agg_kernel(table, src, dst, zeros):
    return _get_agg()(table, src, dst, zeros)


def _detile_body(ei_ref, osrc_ref, odst_ref):
    b = pl.program_id(0)
    e2 = ei_ref[...]
    s = e2[0].reshape(_DR, _CH)
    d = e2[1].reshape(_DR, _CH)
    flat = (b * _DE
            + lax.broadcasted_iota(jnp.int32, (_DR, _CH), 0) * _CH
            + lax.broadcasted_iota(jnp.int32, (_DR, _CH), 1))
    inb = flat < _E
    pad = flat - _E
    osrc_ref[...] = jnp.where(inb, s, pad & 8191)
    odst_ref[...] = jnp.where(inb, d, _N + (pad & 127))


def _detile(edge_index):
    return pl.pallas_call(
        _detile_body,
        grid=(_DK,),
        in_specs=[pl.BlockSpec((2, _DE), lambda b: (0, b))],
        out_specs=[pl.BlockSpec((_DR, _CH), lambda b: (b, 0))] * 2,
        out_shape=[jax.ShapeDtypeStruct((_NROW, _CH), jnp.int32)] * 2,
    )(edge_index)


def _mm(a, w):
    return jnp.dot(a, w, preferred_element_type=jnp.float32)


def _dense1_body(x_ref, p_ref, w1, b1, w2, b2, w3, b3, o_ref):
    h = x_ref[...] + p_ref[0] + p_ref[1]
    h = jnp.maximum(_mm(h, w1[...]) + b1[...], 0.0)
    h = jnp.maximum(_mm(h, w2[...]) + b2[...], 0.0)
    h = _mm(h, w3[...]) + b3[...]
    o_ref[...] = jnp.maximum(h, 0.0)


def _dense2_body(x_ref, p_ref, w1, b1, w2, b2, w3, b3, w4, b4, fw, fb, o_ref):
    h = x_ref[...] + p_ref[0] + p_ref[1]
    h = jnp.maximum(_mm(h, w1[...]) + b1[...], 0.0)
    h = jnp.maximum(_mm(h, w2[...]) + b2[...], 0.0)
    h = jnp.maximum(_mm(h, w3[...]) + b3[...], 0.0)
    h = _mm(h, w4[...]) + b4[...]
    o_ref[...] = jnp.maximum(_mm(h, fw[...]) + fb[...], 0.0)


def _dense3_body(nblk, x_ref, p_ref, batch_ref,
                 w1, b1, w2, b2, w3, b3, fw2, fb2, fw3, fb3,
                 o_ref, acc_ref):
    k = pl.program_id(0)

    @pl.when(k == 0)
    def _():
        acc_ref[...] = jnp.zeros_like(acc_ref)

    h = x_ref[...] + p_ref[0] + p_ref[1]
    h = jnp.maximum(_mm(h, w1[...]) + b1[...], 0.0)
    h = jnp.maximum(_mm(h, w2[...]) + b2[...], 0.0)
    h = _mm(h, w3[...]) + b3[...]
    r = h.shape[0]
    gi = lax.broadcasted_iota(jnp.int32, (_G, r), 0)
    m = (gi == batch_ref[0]).astype(jnp.float32)
    acc_ref[...] += _mm(m, h)

    @pl.when(k == nblk - 1)
    def _():
        z = jnp.maximum(_mm(acc_ref[...], fw2[...]) + fb2[...], 0.0)
        logits = _mm(z, fw3[...]) + fb3[...]
        mx = jnp.max(logits, axis=1, keepdims=True)
        lse = jnp.log(jnp.sum(jnp.exp(logits - mx), axis=1, keepdims=True)) + mx
        o_ref[...] = logits - lse


_R = 2000
_KB = _N // _R

_wspec = pl.BlockSpec((_D, _D), lambda i: (0, 0))
_bspec = pl.BlockSpec((1, _D), lambda i: (0, 0))
_rspec = pl.BlockSpec((_R, _D), lambda i: (i, 0))
_pspec = pl.BlockSpec((2, _R, _D), lambda i: (0, i, 0))


def _dense1(x, parts, w1, b1, w2, b2, w3, b3):
    return pl.pallas_call(
        _dense1_body,
        grid=(_KB,),
        in_specs=[_rspec, _pspec, _wspec, _bspec, _wspec, _bspec, _wspec, _bspec],
        out_specs=_rspec,
        out_shape=jax.ShapeDtypeStruct((_N, _D), jnp.float32),
    )(x, parts, w1, b1, w2, b2, w3, b3)


def _dense2(x, parts, w1, b1, w2, b2, w3, b3, w4, b4, fw, fb):
    return pl.pallas_call(
        _dense2_body,
        grid=(_KB,),
        in_specs=[_rspec, _pspec] + [_wspec, _bspec] * 5,
        out_specs=_rspec,
        out_shape=jax.ShapeDtypeStruct((_N, _D), jnp.float32),
    )(x, parts, w1, b1, w2, b2, w3, b3, w4, b4, fw, fb)


def _dense3(x, parts, batch3, w1, b1, w2, b2, w3, b3, fw2, fb2, fw3p, fb3p):
    return pl.pallas_call(
        functools.partial(_dense3_body, _KB),
        grid=(_KB,),
        in_specs=[_rspec, _pspec, pl.BlockSpec((1, 1, _R), lambda i: (i, 0, 0))]
                 + [_wspec, _bspec] * 5,
        out_specs=pl.BlockSpec((_G, _D), lambda i: (0, 0)),
        out_shape=jax.ShapeDtypeStruct((_G, _D), jnp.float32),
        scratch_shapes=[pltpu.VMEM((_G, _D), jnp.float32)],
    )(x, parts, batch3, w1, b1, w2, b2, w3, b3, fw2, fb2, fw3p, fb3p)


def kernel(x, edge_index, batch,
           c1w1, c1b1, c1w2, c1b2, c1w3, c1b3,
           c2w1, c2b1, c2w2, c2b2, c2w3, c2b3, c2w4, c2b4,
           c3w1, c3b1, c3w2, c3b2, c3w3, c3b3,
           fc1w, fc1b, fc2w, fc2b, fc3w, fc3b):
    src, dst = _detile(edge_index)
    zeros = jnp.asarray(np.zeros((_NACC, _D), np.float32))
    batch3 = batch.reshape(_KB, 1, _R)

    def r2(b):
        return b.reshape(1, _D)

    fw3p = jnp.zeros((_D, _D), jnp.float32).at[:, :_OUT].set(fc3w)
    fb3p = jnp.full((1, _D), -1e9, jnp.float32).at[0, :_OUT].set(fc3b)

    p1 = _agg_kernel(x, src, dst, zeros)
    h1 = _dense1(x, p1, c1w1, r2(c1b1), c1w2, r2(c1b2), c1w3, r2(c1b3))
    p2 = _agg_kernel(h1, src, dst, zeros)
    h2 = _dense2(h1, p2, c2w1, r2(c2b1), c2w2, r2(c2b2), c2w3, r2(c2b3),
                 c2w4, r2(c2b4), fc1w, r2(fc1b))
    p3 = _agg_kernel(h2, src, dst, zeros)
    res = _dense3(h2, p3, batch3, c3w1, r2(c3b1), c3w2, r2(c3b2),
                  c3w3, r2(c3b3), fc2w, r2(fc2b), fw3p, fb3p)
    return res[:, :_OUT]
